# baseline (speedup 1.0000x reference)
import functools

import jax
import jax.numpy as jnp
from jax import lax
from jax.experimental import pallas as pl
from jax.experimental.pallas import tpu as pltpu
from jax.experimental.pallas import tpu_sc as plsc

B = 4096
L = 200
F = 4
ROW_W = L * F
NC, NS = 2, 16
NW = NC * NS
RPW = B // NW
R = 32
NCHUNK = RPW // R
CH_IN = R * ROW_W
CH_OUT = R * L
NGRP = (L + 15) // 16
WIDTHS = (12, 31, 7, 24)

_mesh = plsc.VectorSubcoreMesh(core_axis_name="c", subcore_axis_name="s")


@functools.partial(
    pl.kernel,
    out_type=jax.ShapeDtypeStruct((B * L,), jnp.float32),
    mesh=_mesh,
    compiler_params=pltpu.CompilerParams(needs_layout_passes=False),
    scratch_types=[
        pltpu.VMEM((CH_IN,), jnp.int32),
        pltpu.VMEM((CH_IN,), jnp.int32),
        pltpu.VMEM((CH_OUT + 16,), jnp.float32),
        pltpu.VMEM((CH_OUT + 16,), jnp.float32),
        pltpu.SemaphoreType.DMA,
        pltpu.SemaphoreType.DMA,
        pltpu.SemaphoreType.DMA,
        pltpu.SemaphoreType.DMA,
    ],
)
def _t2v_sc(x_hbm, out_hbm, in0, in1, ob0, ob1, si0, si1, so0, so1):
    wid = lax.axis_index("s") * NC + lax.axis_index("c")
    in_base = wid * (RPW * ROW_W)
    out_base = wid * (RPW * L)
    inbufs, obufs = (in0, in1), (ob0, ob1)
    isems, osems = (si0, si1), (so0, so1)

    iota = lax.iota(jnp.int32, 16)
    iota4 = iota * 4
    lane_lt8 = iota < 8
    one = jnp.full((16,), 1.0, jnp.float32)
    zero = jnp.full((16,), 0.0, jnp.float32)

    def start_in(c):
        return pltpu.async_copy(
            x_hbm.at[pl.ds(in_base + c * CH_IN, CH_IN)],
            inbufs[c % 2], isems[c % 2])

    def start_out(c):
        return pltpu.async_copy(
            obufs[c % 2].at[pl.ds(0, CH_OUT)],
            out_hbm.at[pl.ds(out_base + c * CH_OUT, CH_OUT)],
            osems[c % 2])

    def process(c):
        ib, ob = inbufs[c % 2], obufs[c % 2]

        def row_body(r, carry):
            rbase_in = r * ROW_W
            rbase_out = r * L
            acc = zero
            for j in range(NGRP):
                gbase = rbase_in + j * 64
                idx0 = iota4 + gbase
                if j == NGRP - 1:
                    idx0 = jnp.where(lane_lt8, idx0, gbase)
                cnt = zero
                for f, w in enumerate(WIDTHS):
                    v = plsc.load_gather(ib, [idx0 + f])
                    vu = plsc.bitcast(v, jnp.uint32)
                    cnt = cnt + jnp.where(vu < jnp.uint32(w), one, zero)
                if j == NGRP - 1:
                    cnt = jnp.where(lane_lt8, cnt, zero)
                acc = acc + cnt * cnt
                ob[pl.ds(rbase_out + j * 16, 16)] = cnt
            t = jnp.full((16,), jnp.sum(acc), jnp.float32)
            gi = jnp.int32(0x5F3759DF) - (plsc.bitcast(t, jnp.int32) >> 1)
            g = plsc.bitcast(gi, jnp.float32)
            for _ in range(3):
                g = g * (1.5 - 0.5 * t * g * g)
            for j in range(NGRP):
                off = rbase_out + j * 16
                ob[pl.ds(off, 16)] = ob[pl.ds(off, 16)] * g
            return carry

        lax.fori_loop(0, R, row_body, 0)

    cp_in = [None] * NCHUNK
    cp_out = [None] * NCHUNK
    cp_in[0] = start_in(0)
    for c in range(NCHUNK):
        if c + 1 < NCHUNK:
            cp_in[c + 1] = start_in(c + 1)
        cp_in[c].wait()
        if c >= 2:
            cp_out[c - 2].wait()
        process(c)
        cp_out[c] = start_out(c)
    cp_out[NCHUNK - 2].wait()
    cp_out[NCHUNK - 1].wait()


def kernel(x):
    x = x.astype(jnp.int32).reshape(B * ROW_W)
    return _t2v_sc(x).reshape(B, L)
